# baseline (speedup 1.0000x reference)
import jax
import jax.numpy as jnp
from jax.experimental import pallas as pl
from jax.experimental.pallas import tpu as pltpu

K = 8192
D = 49
N = 4
BLK = 1024
NB = K // BLK


def _vq_body(x_ref, cb_ref, out_ref, bestv_ref, bestrow_ref):
    i = pl.program_id(0)
    xs = x_ref[...]
    cb = cb_ref[...]
    b2r = jax.lax.dot_general(
        jnp.ones((1, D), jnp.float32), cb * cb, (((1,), (1,)), ((), ())),
        preferred_element_type=jnp.float32)
    dots = jax.lax.dot_general(
        xs, cb, (((1,), (1,)), ((), ())),
        preferred_element_type=jnp.float32)
    dist = b2r - 2.0 * dots
    bv = jnp.min(dist, axis=1, keepdims=True)
    bi = jnp.argmin(dist, axis=1, keepdims=True)
    onehot = (jax.lax.broadcasted_iota(jnp.int32, (N, BLK), 1)
              == bi).astype(jnp.float32)
    rows = jax.lax.dot_general(
        onehot, cb, (((1,), (0,)), ((), ())),
        preferred_element_type=jnp.float32)

    @pl.when(i == 0)
    def _init():
        bestv_ref[...] = bv
        bestrow_ref[...] = rows

    @pl.when(i > 0)
    def _update():
        upd = bv < bestv_ref[...]
        bestv_ref[...] = jnp.where(upd, bv, bestv_ref[...])
        bestrow_ref[...] = jnp.where(upd, rows, bestrow_ref[...])

    @pl.when(i == NB - 1)
    def _emit():
        zq = bestrow_ref[...]
        out_ref[...] = xs + (zq - xs)


def kernel(x, codebook):
    out = pl.pallas_call(
        _vq_body,
        grid=(NB,),
        in_specs=[
            pl.BlockSpec((N, D), lambda i: (0, 0)),
            pl.BlockSpec((BLK, D), lambda i: (i, 0)),
        ],
        out_specs=pl.BlockSpec((N, D), lambda i: (0, 0)),
        out_shape=jax.ShapeDtypeStruct((N, D), jnp.float32),
        scratch_shapes=[
            pltpu.VMEM((N, 1), jnp.float32),
            pltpu.VMEM((N, D), jnp.float32),
        ],
    )(x, codebook)
    return jnp.reshape(out, (4, 7, 7))


# speedup vs baseline: 1.3440x; 1.3440x over previous
import jax
import jax.numpy as jnp
from jax.experimental import pallas as pl
from jax.experimental.pallas import tpu as pltpu

K = 8192
D = 49
N = 4


def _vq_body(x_ref, cb_ref, out_ref):
    xs = x_ref[...]
    cb = cb_ref[...]
    b2r = jax.lax.dot_general(
        jnp.ones((1, D), jnp.float32), cb * cb, (((1,), (1,)), ((), ())),
        preferred_element_type=jnp.float32)
    dots = jax.lax.dot_general(
        xs, cb, (((1,), (1,)), ((), ())),
        preferred_element_type=jnp.float32)
    dist = b2r - 2.0 * dots
    idx = jnp.argmin(dist, axis=1)
    onehot = (jax.lax.broadcasted_iota(jnp.int32, (N, K), 1)
              == idx[:, None]).astype(jnp.float32)
    zq = jax.lax.dot_general(
        onehot, cb, (((1,), (0,)), ((), ())),
        preferred_element_type=jnp.float32)
    out_ref[...] = xs + (zq - xs)


def kernel(x, codebook):
    out = pl.pallas_call(
        _vq_body,
        out_shape=jax.ShapeDtypeStruct((N, D), jnp.float32),
    )(x, codebook)
    return jnp.reshape(out, (4, 7, 7))
